# Initial kernel scaffold; baseline (speedup 1.0000x reference)
import jax
import jax.numpy as jnp
from jax.experimental import pallas as pl
from jax.experimental.pallas import tpu as pltpu

D_SINGLE = 384
D_PAIR = 128
NUM_AA = 21
MAX_REL = 32
N_RES = 1024

BI = 8
PAD_ROWS = 2 * N_RES + 2 * MAX_REL


def _pair_kernel(table_ref, out_ref, padded_ref):
    blk = pl.program_id(0)

    @pl.when(blk == 0)
    def _build_padded():
        hi = table_ref[2 * MAX_REL, :]
        lo = table_ref[0, :]
        padded_ref[pl.ds(0, N_RES), :] = jnp.broadcast_to(hi, (N_RES, D_PAIR))
        padded_ref[pl.ds(N_RES, 2 * MAX_REL + 1), :] = table_ref[::-1, :]
        tail = PAD_ROWS - N_RES - 2 * MAX_REL - 1
        padded_ref[pl.ds(N_RES + 2 * MAX_REL + 1, tail), :] = jnp.broadcast_to(
            lo, (tail, D_PAIR)
        )

    for k in range(BI):
        i = blk * BI + k
        start = (N_RES + MAX_REL) - i
        out_ref[k] = padded_ref[pl.ds(start, N_RES), :]


def _single_kernel(seq_ref, w_ref, b_ref, out_ref):
    seq = seq_ref[:, 0]
    oh = seq[:, None] == jax.lax.broadcasted_iota(jnp.int32, (N_RES, NUM_AA), 1)
    out_ref[...] = (
        jnp.dot(oh.astype(jnp.float32), w_ref[...], preferred_element_type=jnp.float32)
        + b_ref[0, :]
    )


def kernel(target_seq, W_dense, b_dense, relpos_table):
    pair = pl.pallas_call(
        _pair_kernel,
        grid=(N_RES // BI,),
        in_specs=[pl.BlockSpec((2 * MAX_REL + 1, D_PAIR), lambda b: (0, 0))],
        out_specs=pl.BlockSpec((BI, N_RES, D_PAIR), lambda b: (b, 0, 0)),
        out_shape=jax.ShapeDtypeStruct((N_RES, N_RES, D_PAIR), jnp.float32),
        scratch_shapes=[pltpu.VMEM((PAD_ROWS, D_PAIR), jnp.float32)],
    )(relpos_table)

    single = pl.pallas_call(
        _single_kernel,
        in_specs=[
            pl.BlockSpec((N_RES, 1), lambda: (0, 0)),
            pl.BlockSpec((NUM_AA, D_SINGLE), lambda: (0, 0)),
            pl.BlockSpec((1, D_SINGLE), lambda: (0, 0)),
        ],
        out_specs=pl.BlockSpec((N_RES, D_SINGLE), lambda: (0, 0)),
        out_shape=jax.ShapeDtypeStruct((N_RES, D_SINGLE), jnp.float32),
    )(target_seq.astype(jnp.int32).reshape(N_RES, 1), W_dense, b_dense.reshape(1, D_SINGLE))

    return (single, pair)


# speedup vs baseline: 22.8101x; 22.8101x over previous
import jax
import jax.numpy as jnp
from jax.experimental import pallas as pl
from jax.experimental.pallas import tpu as pltpu

D_SINGLE = 384
D_PAIR = 128
NUM_AA = 21
MAX_REL = 32
N_RES = 1024

BI = 8
PAD_ROWS = 2 * N_RES + 2 * MAX_REL


def _pair_kernel(table_ref, out_ref, padded_ref):
    blk = pl.program_id(0)

    @pl.when(blk == 0)
    def _build_padded():
        hi = table_ref[2 * MAX_REL, :]
        lo = table_ref[0, :]
        padded_ref[pl.ds(0, N_RES), :] = jnp.broadcast_to(hi, (N_RES, D_PAIR))
        for r in range(2 * MAX_REL + 1):
            padded_ref[N_RES + r, :] = table_ref[2 * MAX_REL - r, :]
        tail = PAD_ROWS - N_RES - 2 * MAX_REL - 1
        padded_ref[pl.ds(N_RES + 2 * MAX_REL + 1, tail), :] = jnp.broadcast_to(
            lo, (tail, D_PAIR)
        )

    for k in range(BI):
        i = blk * BI + k
        start = (N_RES + MAX_REL) - i
        out_ref[k] = padded_ref[pl.ds(start, N_RES), :]


def _single_kernel(seq_ref, w_ref, b_ref, out_ref):
    seq = seq_ref[:, 0]
    oh = seq[:, None] == jax.lax.broadcasted_iota(jnp.int32, (N_RES, NUM_AA), 1)
    out_ref[...] = (
        jnp.dot(oh.astype(jnp.float32), w_ref[...], preferred_element_type=jnp.float32)
        + b_ref[0, :]
    )


def kernel(target_seq, W_dense, b_dense, relpos_table):
    pair = pl.pallas_call(
        _pair_kernel,
        grid=(N_RES // BI,),
        in_specs=[pl.BlockSpec((2 * MAX_REL + 1, D_PAIR), lambda b: (0, 0))],
        out_specs=pl.BlockSpec((BI, N_RES, D_PAIR), lambda b: (b, 0, 0)),
        out_shape=jax.ShapeDtypeStruct((N_RES, N_RES, D_PAIR), jnp.float32),
        scratch_shapes=[pltpu.VMEM((PAD_ROWS, D_PAIR), jnp.float32)],
    )(relpos_table)

    single = pl.pallas_call(
        _single_kernel,
        in_specs=[
            pl.BlockSpec((N_RES, 1), lambda: (0, 0)),
            pl.BlockSpec((NUM_AA, D_SINGLE), lambda: (0, 0)),
            pl.BlockSpec((1, D_SINGLE), lambda: (0, 0)),
        ],
        out_specs=pl.BlockSpec((N_RES, D_SINGLE), lambda: (0, 0)),
        out_shape=jax.ShapeDtypeStruct((N_RES, D_SINGLE), jnp.float32),
    )(target_seq.astype(jnp.int32).reshape(N_RES, 1), W_dense, b_dense.reshape(1, D_SINGLE))

    return (single, pair)
